# baseline (speedup 1.0000x reference)
import jax
import jax.numpy as jnp
import numpy as np
from jax.experimental import pallas as pl
from jax.experimental.pallas import tpu as pltpu

_ZONE_COLS = [
    np.array([1, 3, 4, 7, 8, 9, 10, 11, 16, 17, 18, 19, 20, 21]) - 1,
    np.array([17, 18, 19, 20, 21, 27, 28, 29, 30, 31, 36, 37, 38, 39, 40, 41]) - 1,
    np.array([37, 38, 39, 40, 41, 47, 48, 49, 50, 51]) - 1,
    np.array([56, 57, 58, 59, 62, 63]) - 1,
    np.array([59, 60, 61, 63, 64]) - 1,
    np.array([41, 42, 43, 44, 45, 46, 51, 52, 53, 54, 55, 56]) - 1,
    np.array([21, 22, 23, 24, 25, 31, 32, 33, 34, 35, 41, 42, 43, 44, 45, 46]) - 1,
    np.array([2, 5, 6, 11, 12, 13, 14, 15, 21, 22, 23, 24, 25, 26]) - 1,
]
_WIDTHS = [len(z) for z in _ZONE_COLS]

_N_ROWS = 16384
_N_COLS = 64
_BLOCK_ROWS = 8192


def _runs(cols):
    out = []
    start = int(cols[0])
    length = 1
    for a, b in zip(cols[:-1], cols[1:]):
        if int(b) == int(a) + 1:
            length += 1
        else:
            out.append((start, length))
            start = int(b)
            length = 1
    out.append((start, length))
    return out


_RUNS = [_runs(z) for z in _ZONE_COLS]


def _split_body(in_ref, *out_refs):
    x2 = in_ref[...]
    x = jnp.stack([x2[:, :_N_COLS], x2[:, _N_COLS:]], axis=1).reshape(
        _BLOCK_ROWS, _N_COLS
    )
    xt = x.T
    for z, runs in enumerate(_RUNS):
        out_refs[z][...] = jnp.concatenate(
            [xt[a : a + l, :] for (a, l) in runs], axis=0
        )


@jax.jit
def kernel(inputs):
    grid = (_N_ROWS // _BLOCK_ROWS,)
    outs_t = pl.pallas_call(
        _split_body,
        grid=grid,
        in_specs=[
            pl.BlockSpec((_BLOCK_ROWS // 2, 2 * _N_COLS), lambda i: (i, 0))
        ],
        out_specs=[
            pl.BlockSpec((w, _BLOCK_ROWS), lambda i: (0, i)) for w in _WIDTHS
        ],
        out_shape=tuple(
            jax.ShapeDtypeStruct((w, _N_ROWS), jnp.float32) for w in _WIDTHS
        ),
        compiler_params=pltpu.CompilerParams(
            dimension_semantics=("arbitrary",),
        ),
    )(inputs.reshape(_N_ROWS // 2, 2 * _N_COLS))
    return tuple(o.T for o in outs_t)


# speedup vs baseline: 2.1934x; 2.1934x over previous
import jax
import jax.numpy as jnp
import numpy as np
from jax.experimental import pallas as pl
from jax.experimental.pallas import tpu as pltpu

_ZONE_COLS = [
    np.array([1, 3, 4, 7, 8, 9, 10, 11, 16, 17, 18, 19, 20, 21]) - 1,
    np.array([17, 18, 19, 20, 21, 27, 28, 29, 30, 31, 36, 37, 38, 39, 40, 41]) - 1,
    np.array([37, 38, 39, 40, 41, 47, 48, 49, 50, 51]) - 1,
    np.array([56, 57, 58, 59, 62, 63]) - 1,
    np.array([59, 60, 61, 63, 64]) - 1,
    np.array([41, 42, 43, 44, 45, 46, 51, 52, 53, 54, 55, 56]) - 1,
    np.array([21, 22, 23, 24, 25, 31, 32, 33, 34, 35, 41, 42, 43, 44, 45, 46]) - 1,
    np.array([2, 5, 6, 11, 12, 13, 14, 15, 21, 22, 23, 24, 25, 26]) - 1,
]
_WIDTHS = [len(z) for z in _ZONE_COLS]

_N_ROWS = 16384
_N_COLS = 64
_BLOCK_ROWS = 8192


def _runs(cols):
    out = []
    start = int(cols[0])
    length = 1
    for a, b in zip(cols[:-1], cols[1:]):
        if int(b) == int(a) + 1:
            length += 1
        else:
            out.append((start, length))
            start = int(b)
            length = 1
    out.append((start, length))
    return out


_RUNS = [_runs(z) for z in _ZONE_COLS]


def _split_body(in_ref, *out_refs):
    xt = in_ref[...].T
    for z, runs in enumerate(_RUNS):
        out_refs[z][...] = jnp.concatenate(
            [xt[a : a + l, :] for (a, l) in runs], axis=0
        )


@jax.jit
def kernel(inputs):
    grid = (_N_ROWS // _BLOCK_ROWS,)
    outs_t = pl.pallas_call(
        _split_body,
        grid=grid,
        in_specs=[pl.BlockSpec((_BLOCK_ROWS, _N_COLS), lambda i: (i, 0))],
        out_specs=[
            pl.BlockSpec((w, _BLOCK_ROWS), lambda i: (0, i)) for w in _WIDTHS
        ],
        out_shape=tuple(
            jax.ShapeDtypeStruct((w, _N_ROWS), jnp.float32) for w in _WIDTHS
        ),
        compiler_params=pltpu.CompilerParams(
            dimension_semantics=("arbitrary",),
        ),
    )(inputs)
    return tuple(o.T for o in outs_t)
